# baseline (speedup 1.0000x reference)
import functools

import jax
import jax.numpy as jnp
from jax import lax
from jax.experimental import pallas as pl
from jax.experimental.pallas import tpu as pltpu
from jax.experimental.pallas import tpu_sc as plsc

_HID = 64
_B, _S = 1024, 200
_NC, _NS = 2, 16
_NW = _NC * _NS
_CHUNK = 128
_NBBLK = _B // _CHUNK
_SPERW = _S // (_NW // _NBBLK)
_NBUF = 5
_NG = _SPERW // _NBUF
_LANE = 16


def _sc_embed_body(ids_hbm, cw_hbm, table_hbm, out_hbm,
                   idx_v, g0, g1, g2, g3, g4, s0_, s1_, s2_, s3_, s4_,
                   cw_v, gsem, ssem):
    gbufs = [g0, g1, g2, g3, g4]
    sbufs = [s0_, s1_, s2_, s3_, s4_]
    wid = lax.axis_index("s") * _NC + lax.axis_index("c")
    cb = wid % _NBBLK
    s0 = (wid // _NBBLK) * _SPERW

    pltpu.sync_copy(ids_hbm.at[pl.ds(s0, _SPERW), pl.ds(cb, 1)], idx_v)
    pltpu.sync_copy(cw_hbm, cw_v)

    w_vec = cw_v[pl.ds(_HID, _LANE)]
    c_vecs = [cw_v[pl.ds(g * _LANE, _LANE)] for g in range(4)]
    hidx = [lax.iota(jnp.int32, 16) + (g * _LANE) for g in range(4)]
    ridx_hi = [h >> 3 for h in hidx]
    ridx_lo = [h & 7 for h in hidx]

    def gather_start(k, b):
        pltpu.make_async_copy(
            table_hbm.at[idx_v.at[k, 0]], gbufs[b], gsem.at[b]).start()

    def gather_wait(b):
        pltpu.make_async_copy(
            table_hbm.at[idx_v.at[0, 0]], gbufs[b], gsem.at[b]).wait()

    def store_start(k, b):
        pltpu.make_async_copy(
            sbufs[b].at[:, :, pl.ds(0, _CHUNK)],
            out_hbm.at[s0 + k, slice(None), cb], ssem.at[b]).start()

    def store_wait(b):
        pltpu.make_async_copy(
            sbufs[b].at[:, :, pl.ds(0, _CHUNK)],
            out_hbm.at[s0, slice(None), cb], ssem.at[b]).wait()

    def fma_transpose(b):
        gb = gbufs[b]
        sb = sbufs[b]

        @plsc.parallel_loop(0, _CHUNK // 4, unroll=2)
        def _body(t4):
            for dt in range(4):
                t = t4 * 4 + dt
                lv = jnp.full((16,), t, jnp.int32)
                for g in range(4):
                    v = gb[t, pl.ds(g * _LANE, _LANE)] * w_vec + c_vecs[g]
                    plsc.store_scatter(sb, [ridx_hi[g], ridx_lo[g], lv], v)

    for b in range(_NBUF):
        gather_start(b, b)

    def outer(g, carry):
        for b in range(_NBUF):
            k = g * _NBUF + b
            gather_wait(b)

            @pl.when(g > 0)
            def _wait_prev_store():
                store_wait(b)

            fma_transpose(b)
            store_start(k, b)

            @pl.when(g < _NG - 1)
            def _refill():
                gather_start(k + _NBUF, b)
        return carry

    lax.fori_loop(0, _NG, outer, 0, unroll=False)

    for b in range(_NBUF):
        store_wait(b)


_embed_call = functools.partial(
    pl.kernel,
    out_type=jax.ShapeDtypeStruct((_S, 8, _NBBLK, 8, _CHUNK), jnp.float32),
    mesh=plsc.VectorSubcoreMesh(core_axis_name="c", subcore_axis_name="s"),
    compiler_params=pltpu.CompilerParams(
        use_tc_tiling_on_sc=False, needs_layout_passes=False),
    scratch_types=[
        pltpu.VMEM((_SPERW, 1, _CHUNK), jnp.int32),
        *[pltpu.VMEM((_CHUNK, _HID), jnp.float32) for _ in range(_NBUF)],
        *[pltpu.VMEM((8, 8, _CHUNK + 5), jnp.float32) for _ in range(_NBUF)],
        pltpu.VMEM((_HID + _LANE,), jnp.float32),
        pltpu.SemaphoreType.DMA((_NBUF,)),
        pltpu.SemaphoreType.DMA((_NBUF,)),
    ],
)(_sc_embed_body)


_NTILE = 7813
_CFULL = 7808
_CPW = _CFULL // _NW
_PAD = 133


def _sc_convert_body(tabT_hbm, tail_hbm, lin_hbm, stgA, stgB, outA, outB,
                     gsem, ssem):
    wid = lax.axis_index("s") * _NC + lax.axis_index("c")
    base = wid * _CPW
    stgs = [stgA, stgB]
    outs = [outA, outB]

    ivec = lax.iota(jnp.int32, 16)
    dh_v = ivec >> 2
    dl_v = ivec & 3
    drow_v = dl_v >> 1
    dcol_v = (dl_v & 1) * _HID

    def fire8(c, b):
        for r in range(8):
            pltpu.make_async_copy(
                tabT_hbm.at[pl.ds(8 * r, 8), pl.ds(128 * c, 128)],
                stgs[b].at[r, :, pl.ds(0, 128)], gsem.at[b]).start()

    def wait8(b):
        for r in range(8):
            pltpu.make_async_copy(
                tabT_hbm.at[pl.ds(0, 8), pl.ds(0, 128)],
                stgs[b].at[r, :, pl.ds(0, 128)], gsem.at[b]).wait()

    def store_start(c, b):
        pltpu.make_async_copy(
            outs[b].at[:, pl.ds(0, 128)],
            lin_hbm.at[pl.ds(_HID * c, _HID)], ssem.at[b]).start()

    def store_wait(b):
        pltpu.make_async_copy(
            outs[b].at[:, pl.ds(0, 128)],
            lin_hbm.at[pl.ds(0, _HID)], ssem.at[b]).wait()

    def transpose(b, nl):
        stg = stgs[b]
        ob = outs[b]

        @plsc.parallel_loop(0, 8, unroll=2)
        def _body(r_hi):
            rv = jnp.full((16,), r_hi, jnp.int32)
            for s4 in (0, 4):
                sv = s4 + dh_v
                hlo = r_hi * 8 + s4
                for l0 in range(0, nl, 4):
                    v = plsc.load_gather(stg, [rv, sv, l0 + dl_v])
                    plsc.store_scatter(
                        ob, [(l0 >> 1) + drow_v, dcol_v + (hlo + dh_v)], v)

    fire8(base, 0)
    fire8(base + 1, 1)

    def outer(g, carry):
        for b in range(2):
            j = g * 2 + b
            wait8(b)

            @pl.when(g > 0)
            def _wsp():
                store_wait(b)

            transpose(b, 128)
            store_start(base + j, b)

            @pl.when(g < _CPW // 2 - 1)
            def _refill():
                fire8(base + j + 2, b)
        return carry

    lax.fori_loop(0, _CPW // 2, outer, 0, unroll=False)
    for b in range(2):
        store_wait(b)

    @pl.when(wid < 4)
    def _tail_full():
        c = _CFULL + wid
        fire8(c, 0)
        wait8(0)
        transpose(0, 128)
        store_start(c, 0)
        store_wait(0)

    @pl.when(wid == 4)
    def _tail_partial():
        pltpu.sync_copy(tail_hbm, outA.at[pl.ds(0, 32), pl.ds(0, 128)])
        pltpu.sync_copy(outA.at[pl.ds(0, 32), pl.ds(0, 128)],
                        lin_hbm.at[pl.ds(_HID * 7812, 32)])


_convert_call = functools.partial(
    pl.kernel,
    out_type=jax.ShapeDtypeStruct((500000, 128), jnp.float32),
    mesh=plsc.VectorSubcoreMesh(core_axis_name="c", subcore_axis_name="s"),
    compiler_params=pltpu.CompilerParams(needs_layout_passes=False),
    scratch_types=[
        pltpu.VMEM((8, 8, 136), jnp.float32),
        pltpu.VMEM((8, 8, 136), jnp.float32),
        pltpu.VMEM((_HID, _PAD), jnp.float32),
        pltpu.VMEM((_HID, _PAD), jnp.float32),
        pltpu.SemaphoreType.DMA((2,)),
        pltpu.SemaphoreType.DMA((2,)),
    ],
)(_sc_convert_body)


def kernel(input_features, word_table, tt_table, singleton_weight, singleton_bias):
    ids = input_features.T.reshape(_S, _NBBLK, _CHUNK).astype(jnp.int32)
    w = singleton_weight[0].astype(jnp.float32)
    c = tt_table[0].astype(jnp.float32) * w + singleton_bias[0].astype(jnp.float32)
    cw = jnp.concatenate([c, jnp.full((_LANE,), w, jnp.float32)])
    wt = word_table.astype(jnp.float32)
    lin = _convert_call(wt.T, wt[128 * 7812:].reshape(32, 128))
    out5 = _embed_call(ids, cw, lin.reshape(1000000, _HID))
    return out5.transpose(2, 4, 0, 1, 3).reshape(_B, _S, _HID)


# speedup vs baseline: 2.2642x; 2.2642x over previous
import functools

import jax
import jax.numpy as jnp
from jax import lax
from jax.experimental import pallas as pl
from jax.experimental.pallas import tpu as pltpu
from jax.experimental.pallas import tpu_sc as plsc

_HID = 64
_B, _S = 1024, 200
_NC, _NS = 2, 16
_NW = _NC * _NS
_CHUNK = 128
_NBBLK = _B // _CHUNK
_SPERW = _S // (_NW // _NBBLK)
_NBUF = 5
_NG = _SPERW // _NBUF
_LANE = 16


def _sc_embed_body(ids_hbm, cw_hbm, table_hbm, out_hbm,
                   idx_v, g0, g1, g2, g3, g4, s0_, s1_, s2_, s3_, s4_,
                   cw_v, gsem, ssem):
    gbufs = [g0, g1, g2, g3, g4]
    sbufs = [s0_, s1_, s2_, s3_, s4_]
    wid = lax.axis_index("s") * _NC + lax.axis_index("c")
    cb = wid % _NBBLK
    s0 = (wid // _NBBLK) * _SPERW

    pltpu.sync_copy(ids_hbm.at[pl.ds(s0, _SPERW), pl.ds(cb, 1)], idx_v)
    pltpu.sync_copy(cw_hbm, cw_v)

    w_vec = cw_v[pl.ds(_HID, _LANE)]
    c_vecs = [cw_v[pl.ds(g * _LANE, _LANE)] for g in range(4)]
    hidx = [lax.iota(jnp.int32, 16) + (g * _LANE) for g in range(4)]
    ridx_hi = [h >> 3 for h in hidx]
    ridx_lo = [h & 7 for h in hidx]

    def gather_start(k, b):
        pltpu.make_async_copy(
            table_hbm.at[idx_v.at[k, 0]], gbufs[b], gsem.at[b]).start()

    def gather_wait(b):
        pltpu.make_async_copy(
            table_hbm.at[idx_v.at[0, 0]], gbufs[b], gsem.at[b]).wait()

    def store_start(k, b):
        pltpu.make_async_copy(
            sbufs[b].at[:, :, pl.ds(0, _CHUNK)],
            out_hbm.at[s0 + k, slice(None), cb], ssem.at[b]).start()

    def store_wait(b):
        pltpu.make_async_copy(
            sbufs[b].at[:, :, pl.ds(0, _CHUNK)],
            out_hbm.at[s0, slice(None), cb], ssem.at[b]).wait()

    def fma_transpose(b):
        gb = gbufs[b]
        sb = sbufs[b]

        @plsc.parallel_loop(0, _CHUNK // 4, unroll=2)
        def _body(t4):
            for dt in range(4):
                t = t4 * 4 + dt
                lv = jnp.full((16,), t, jnp.int32)
                for g in range(4):
                    v = gb[t, pl.ds(g * _LANE, _LANE)] * w_vec + c_vecs[g]
                    plsc.store_scatter(sb, [ridx_hi[g], ridx_lo[g], lv], v)

    for b in range(_NBUF):
        gather_start(b, b)

    def outer(g, carry):
        for b in range(_NBUF):
            k = g * _NBUF + b
            gather_wait(b)

            @pl.when(g > 0)
            def _wait_prev_store():
                store_wait(b)

            fma_transpose(b)
            store_start(k, b)

            @pl.when(g < _NG - 1)
            def _refill():
                gather_start(k + _NBUF, b)
        return carry

    lax.fori_loop(0, _NG, outer, 0, unroll=False)

    for b in range(_NBUF):
        store_wait(b)


_embed_call = functools.partial(
    pl.kernel,
    out_type=jax.ShapeDtypeStruct((_S, 8, _NBBLK, 8, _CHUNK), jnp.float32),
    mesh=plsc.VectorSubcoreMesh(core_axis_name="c", subcore_axis_name="s"),
    compiler_params=pltpu.CompilerParams(
        use_tc_tiling_on_sc=False, needs_layout_passes=False),
    scratch_types=[
        pltpu.VMEM((_SPERW, 1, _CHUNK), jnp.int32),
        *[pltpu.VMEM((_CHUNK, _HID), jnp.float32) for _ in range(_NBUF)],
        *[pltpu.VMEM((8, 8, _CHUNK + 5), jnp.float32) for _ in range(_NBUF)],
        pltpu.VMEM((_HID + _LANE,), jnp.float32),
        pltpu.SemaphoreType.DMA((_NBUF,)),
        pltpu.SemaphoreType.DMA((_NBUF,)),
    ],
)(_sc_embed_body)


_NTILE = 7813
_PPW = 122


def _sc_convert_body(tabT_hbm, tail_hbm, lin_hbm, stgA, stgB,
                     outA, outB, gsem, ssem):
    wid = lax.axis_index("s") * _NC + lax.axis_index("c")
    base = wid * _PPW
    stgs = [stgA, stgB]
    outs = [outA, outB]

    ivec = lax.iota(jnp.int32, 16)
    dh_v = ivec >> 2
    dl_v = ivec & 3
    drow_v = dl_v >> 1
    dcol_v = (dl_v & 1) * _HID

    def fire8(p, b):
        for r in range(8):
            pltpu.make_async_copy(
                tabT_hbm.at[pl.ds(8 * r, 8), pl.ds(256 * p, 256)],
                stgs[b].at[r, :, pl.ds(0, 256)], gsem.at[b]).start()

    def wait8(b):
        for r in range(8):
            pltpu.make_async_copy(
                tabT_hbm.at[pl.ds(0, 8), pl.ds(0, 256)],
                stgs[b].at[r, :, pl.ds(0, 256)], gsem.at[b]).wait()

    def store_start(p, b):
        pltpu.make_async_copy(
            outs[b].at[:, pl.ds(0, 128)],
            lin_hbm.at[pl.ds(128 * p, 128)], ssem.at[b]).start()

    def store_wait(b):
        pltpu.make_async_copy(
            outs[b].at[:, pl.ds(0, 128)],
            lin_hbm.at[pl.ds(0, 128)], ssem.at[b]).wait()

    def transpose(b, nl):
        stg = stgs[b]
        ob = outs[b]

        @plsc.parallel_loop(0, 8, unroll=1)
        def _body(r_hi):
            rv = jnp.full((16,), r_hi, jnp.int32)

            @plsc.parallel_loop(0, nl, 4, unroll=4)
            def _inner(l0):
                for s4 in (0, 4):
                    sv = s4 + dh_v
                    hlo = r_hi * 8 + s4
                    v = plsc.load_gather(stg, [rv, sv, l0 + dl_v])
                    plsc.store_scatter(
                        ob, [(l0 >> 1) + drow_v, dcol_v + (hlo + dh_v)], v)

    fire8(base, 0)
    fire8(base + 1, 1)

    def outer(g, carry):
        for b in range(2):
            j = g * 2 + b
            wait8(b)

            @pl.when(g > 0)
            def _wsp():
                store_wait(b)

            transpose(b, 256)
            store_start(base + j, b)

            @pl.when(g < _PPW // 2 - 1)
            def _refill():
                fire8(base + j + 2, b)
        return carry

    lax.fori_loop(0, _PPW // 2, outer, 0, unroll=False)
    for b in range(2):
        store_wait(b)

    @pl.when(wid < 2)
    def _tail_full():
        p = _NW * _PPW + wid
        fire8(p, 0)
        wait8(0)
        transpose(0, 256)
        store_start(p, 0)
        store_wait(0)

    @pl.when(wid == 2)
    def _tail_partial():
        pltpu.sync_copy(tail_hbm, outA.at[pl.ds(0, 32), pl.ds(0, 128)])
        pltpu.sync_copy(outA.at[pl.ds(0, 32), pl.ds(0, 128)],
                        lin_hbm.at[pl.ds(_HID * 7812, 32)])


_convert_call = functools.partial(
    pl.kernel,
    out_type=jax.ShapeDtypeStruct((500000, 128), jnp.float32),
    mesh=plsc.VectorSubcoreMesh(core_axis_name="c", subcore_axis_name="s"),
    compiler_params=pltpu.CompilerParams(needs_layout_passes=False),
    scratch_types=[
        *[pltpu.VMEM((8, 8, 264), jnp.float32) for _ in range(2)],
        *[pltpu.VMEM((128, 136), jnp.float32) for _ in range(2)],
        pltpu.SemaphoreType.DMA((2,)),
        pltpu.SemaphoreType.DMA((2,)),
    ],
)(_sc_convert_body)


def kernel(input_features, word_table, tt_table, singleton_weight, singleton_bias):
    ids = input_features.T.reshape(_S, _NBBLK, _CHUNK).astype(jnp.int32)
    w = singleton_weight[0].astype(jnp.float32)
    c = tt_table[0].astype(jnp.float32) * w + singleton_bias[0].astype(jnp.float32)
    cw = jnp.concatenate([c, jnp.full((_LANE,), w, jnp.float32)])
    wt = word_table.astype(jnp.float32)
    lin = _convert_call(wt.T, wt[128 * 7812:].reshape(32, 128))
    out5 = _embed_call(ids, cw, lin.reshape(1000000, _HID))
    return out5.transpose(2, 4, 0, 1, 3).reshape(_B, _S, _HID)


# speedup vs baseline: 2.3055x; 1.0182x over previous
import functools

import jax
import jax.numpy as jnp
from jax import lax
from jax.experimental import pallas as pl
from jax.experimental.pallas import tpu as pltpu
from jax.experimental.pallas import tpu_sc as plsc

_HID = 64
_B, _S = 1024, 200
_NC, _NS = 2, 16
_NW = _NC * _NS
_CHUNK = 128
_NBBLK = _B // _CHUNK
_SPERW = _S // (_NW // _NBBLK)
_NBUF = 5
_NG = _SPERW // _NBUF
_LANE = 16


def _sc_embed_body(ids_hbm, cw_hbm, table_hbm, out_hbm,
                   idx_v, g0, g1, g2, g3, g4, s0_, s1_, s2_, s3_, s4_,
                   cw_v, gsem, ssem):
    gbufs = [g0, g1, g2, g3, g4]
    sbufs = [s0_, s1_, s2_, s3_, s4_]
    wid = lax.axis_index("s") * _NC + lax.axis_index("c")
    cb = wid % _NBBLK
    s0 = (wid // _NBBLK) * _SPERW

    pltpu.sync_copy(ids_hbm.at[pl.ds(s0, _SPERW), pl.ds(cb, 1)], idx_v)
    pltpu.sync_copy(cw_hbm, cw_v)

    w_vec = cw_v[pl.ds(_HID, _LANE)]
    c_vecs = [cw_v[pl.ds(g * _LANE, _LANE)] for g in range(4)]
    hidx = [lax.iota(jnp.int32, 16) + (g * _LANE) for g in range(4)]
    ridx_hi = [h >> 3 for h in hidx]
    ridx_lo = [h & 7 for h in hidx]

    def gather_start(k, b):
        pltpu.make_async_copy(
            table_hbm.at[idx_v.at[k, 0]], gbufs[b], gsem.at[b]).start()

    def gather_wait(b):
        pltpu.make_async_copy(
            table_hbm.at[idx_v.at[0, 0]], gbufs[b], gsem.at[b]).wait()

    def store_start(k, b):
        pltpu.make_async_copy(
            sbufs[b].at[:, :, pl.ds(0, _CHUNK)],
            out_hbm.at[s0 + k, slice(None), cb], ssem.at[b]).start()

    def store_wait(b):
        pltpu.make_async_copy(
            sbufs[b].at[:, :, pl.ds(0, _CHUNK)],
            out_hbm.at[s0, slice(None), cb], ssem.at[b]).wait()

    def fma_transpose(b):
        gb = gbufs[b]
        sb = sbufs[b]

        @plsc.parallel_loop(0, _CHUNK // 4, unroll=2)
        def _body(t4):
            for dt in range(4):
                t = t4 * 4 + dt
                lv = jnp.full((16,), t, jnp.int32)
                for g in range(4):
                    v = gb[t, pl.ds(g * _LANE, _LANE)] * w_vec + c_vecs[g]
                    plsc.store_scatter(sb, [ridx_hi[g], ridx_lo[g], lv], v)

    for b in range(_NBUF):
        gather_start(b, b)

    def outer(g, carry):
        for b in range(_NBUF):
            k = g * _NBUF + b
            gather_wait(b)

            @pl.when(g > 0)
            def _wait_prev_store():
                store_wait(b)

            fma_transpose(b)
            store_start(k, b)

            @pl.when(g < _NG - 1)
            def _refill():
                gather_start(k + _NBUF, b)
        return carry

    lax.fori_loop(0, _NG, outer, 0, unroll=False)

    for b in range(_NBUF):
        store_wait(b)


_embed_call = functools.partial(
    pl.kernel,
    out_type=jax.ShapeDtypeStruct((_S, 8, _NBBLK, 8, _CHUNK), jnp.float32),
    mesh=plsc.VectorSubcoreMesh(core_axis_name="c", subcore_axis_name="s"),
    compiler_params=pltpu.CompilerParams(
        use_tc_tiling_on_sc=False, needs_layout_passes=False),
    scratch_types=[
        pltpu.VMEM((_SPERW, 1, _CHUNK), jnp.int32),
        *[pltpu.VMEM((_CHUNK, _HID), jnp.float32) for _ in range(_NBUF)],
        *[pltpu.VMEM((8, 8, _CHUNK + 5), jnp.float32) for _ in range(_NBUF)],
        pltpu.VMEM((_HID + _LANE,), jnp.float32),
        pltpu.SemaphoreType.DMA((_NBUF,)),
        pltpu.SemaphoreType.DMA((_NBUF,)),
    ],
)(_sc_embed_body)


_NTILE = 7813
_PPW = 122


def _sc_convert_body(tabT_hbm, tail_hbm, lin_hbm, stgA, stgB,
                     outA, outB, gsem, ssem):
    wid = lax.axis_index("s") * _NC + lax.axis_index("c")
    base = wid * _PPW
    stgs = [stgA, stgB]
    outs = [outA, outB]

    ivec = lax.iota(jnp.int32, 16)
    dh_v = ivec >> 2
    dl_v = ivec & 3
    drow_v = dl_v >> 1
    dcol_v = (dl_v & 1) * _HID

    def fire8(p, b):
        for r in range(8):
            pltpu.make_async_copy(
                tabT_hbm.at[pl.ds(8 * r, 8), pl.ds(256 * p, 256)],
                stgs[b].at[r, :, pl.ds(0, 256)], gsem.at[b]).start()

    def wait8(b):
        for r in range(8):
            pltpu.make_async_copy(
                tabT_hbm.at[pl.ds(0, 8), pl.ds(0, 256)],
                stgs[b].at[r, :, pl.ds(0, 256)], gsem.at[b]).wait()

    def store_start(p, b):
        pltpu.make_async_copy(
            outs[b].at[:, pl.ds(0, 128)],
            lin_hbm.at[pl.ds(128 * p, 128)], ssem.at[b]).start()

    def store_wait(b):
        pltpu.make_async_copy(
            outs[b].at[:, pl.ds(0, 128)],
            lin_hbm.at[pl.ds(0, 128)], ssem.at[b]).wait()

    def transpose(b, nl):
        stg = stgs[b]
        ob = outs[b]

        @plsc.parallel_loop(0, 8, unroll=1)
        def _body(r_hi):
            rv = jnp.full((16,), r_hi, jnp.int32)

            @plsc.parallel_loop(0, nl, 4, unroll=8)
            def _inner(l0):
                for s4 in (0, 4):
                    sv = s4 + dh_v
                    hlo = r_hi * 8 + s4
                    v = plsc.load_gather(stg, [rv, sv, l0 + dl_v])
                    plsc.store_scatter(
                        ob, [(l0 >> 1) + drow_v, dcol_v + (hlo + dh_v)], v)

    fire8(base, 0)
    fire8(base + 1, 1)

    def outer(g, carry):
        for b in range(2):
            j = g * 2 + b
            wait8(b)

            @pl.when(g > 0)
            def _wsp():
                store_wait(b)

            transpose(b, 256)
            store_start(base + j, b)

            @pl.when(g < _PPW // 2 - 1)
            def _refill():
                fire8(base + j + 2, b)
        return carry

    lax.fori_loop(0, _PPW // 2, outer, 0, unroll=False)
    for b in range(2):
        store_wait(b)

    @pl.when(wid < 2)
    def _tail_full():
        p = _NW * _PPW + wid
        fire8(p, 0)
        wait8(0)
        transpose(0, 256)
        store_start(p, 0)
        store_wait(0)

    @pl.when(wid == 2)
    def _tail_partial():
        pltpu.sync_copy(tail_hbm, outA.at[pl.ds(0, 32), pl.ds(0, 128)])
        pltpu.sync_copy(outA.at[pl.ds(0, 32), pl.ds(0, 128)],
                        lin_hbm.at[pl.ds(_HID * 7812, 32)])


_convert_call = functools.partial(
    pl.kernel,
    out_type=jax.ShapeDtypeStruct((500000, 128), jnp.float32),
    mesh=plsc.VectorSubcoreMesh(core_axis_name="c", subcore_axis_name="s"),
    compiler_params=pltpu.CompilerParams(needs_layout_passes=False),
    scratch_types=[
        *[pltpu.VMEM((8, 8, 264), jnp.float32) for _ in range(2)],
        *[pltpu.VMEM((128, 136), jnp.float32) for _ in range(2)],
        pltpu.SemaphoreType.DMA((2,)),
        pltpu.SemaphoreType.DMA((2,)),
    ],
)(_sc_convert_body)


def kernel(input_features, word_table, tt_table, singleton_weight, singleton_bias):
    ids = input_features.T.reshape(_S, _NBBLK, _CHUNK).astype(jnp.int32)
    w = singleton_weight[0].astype(jnp.float32)
    c = tt_table[0].astype(jnp.float32) * w + singleton_bias[0].astype(jnp.float32)
    cw = jnp.concatenate([c, jnp.full((_LANE,), w, jnp.float32)])
    wt = word_table.astype(jnp.float32)
    lin = _convert_call(wt.T, wt[128 * 7812:].reshape(32, 128))
    out5 = _embed_call(ids, cw, lin.reshape(1000000, _HID))
    return out5.transpose(2, 4, 0, 1, 3).reshape(_B, _S, _HID)
